# speedup vs baseline: 1.5326x; 1.5326x over previous
import functools

import jax
import jax.numpy as jnp
from jax import lax
from jax.experimental import pallas as pl
from jax.experimental.pallas import tpu as pltpu
from jax.experimental.pallas import tpu_sc as plsc

HIDDEN = 768
MAX_POS = 2048
EPS = 1e-12

NC = 2
NS = 16
NW = NC * NS

CHUNK = 128

TOK_BLK = 256


def _sc_gather_body(ids_hbm, table_hbm, out_hbm, idx_v, rows_v, sem):
    wid = lax.axis_index("s") * NC + lax.axis_index("c")
    n_chunks = ids_hbm.shape[0] // NW
    base_chunk = wid * n_chunks
    pltpu.sync_copy(ids_hbm.at[pl.ds(base_chunk, n_chunks)], idx_v)
    for j in range(n_chunks):
        pltpu.async_copy(table_hbm.at[idx_v.at[j]], rows_v, sem).wait()
        pltpu.sync_copy(
            rows_v, out_hbm.at[pl.ds((base_chunk + j) * CHUNK, CHUNK)])


def _sc_gather(ids_flat, word_emb):
    tokens = ids_flat.shape[0]
    ids2d = ids_flat.reshape(tokens // CHUNK, CHUNK)
    n_chunks = (tokens // CHUNK) // NW
    mesh = plsc.VectorSubcoreMesh(core_axis_name="c", subcore_axis_name="s")
    return pl.kernel(
        _sc_gather_body,
        out_type=jax.ShapeDtypeStruct((tokens, HIDDEN), jnp.float32),
        mesh=mesh,
        scratch_types=[
            pltpu.VMEM((n_chunks, CHUNK), jnp.int32),
            pltpu.VMEM((CHUNK, HIDDEN), jnp.float32),
            pltpu.SemaphoreType.DMA,
        ],
    )(ids2d, word_emb)


def _tc_ln_body(tt_ref, gath_ref, pos_ref, aux_ref, out_ref):
    tt = tt_ref[0][0].reshape(TOK_BLK, 1)
    type0 = aux_ref[0:1, :]
    type1 = aux_ref[1:2, :]
    w = aux_ref[2:3, :]
    b = aux_ref[3:4, :]
    e = gath_ref[...] + pos_ref[...] + jnp.where(tt == 0, type0, type1)
    mean = jnp.mean(e, axis=-1, keepdims=True)
    cen = e - mean
    var = jnp.mean(cen * cen, axis=-1, keepdims=True)
    out_ref[...] = w * (cen / jnp.sqrt(var + EPS)) + b


def _tc_ln(tt_flat, gathered, pos_emb, aux):
    tokens = gathered.shape[0]
    n_blk = tokens // TOK_BLK
    pos_blocks = MAX_POS // TOK_BLK
    tt3d = tt_flat.reshape(n_blk, 1, TOK_BLK)
    return pl.pallas_call(
        _tc_ln_body,
        grid=(n_blk,),
        in_specs=[
            pl.BlockSpec((1, 1, TOK_BLK), lambda i: (i, 0, 0)),
            pl.BlockSpec((TOK_BLK, HIDDEN), lambda i: (i, 0)),
            pl.BlockSpec((TOK_BLK, HIDDEN), lambda i: (i % pos_blocks, 0)),
            pl.BlockSpec((8, HIDDEN), lambda i: (0, 0)),
        ],
        out_specs=pl.BlockSpec((TOK_BLK, HIDDEN), lambda i: (i, 0)),
        out_shape=jax.ShapeDtypeStruct((tokens, HIDDEN), jnp.float32),
    )(tt3d, gathered, pos_emb, aux)


def kernel(input_ids, token_type_ids, word_emb, pos_emb, type_emb, ln_weight,
           ln_bias):
    batch, seq = input_ids.shape
    tokens = batch * seq
    ids_flat = input_ids.reshape(tokens).astype(jnp.int32)
    tt_flat = token_type_ids.reshape(tokens).astype(jnp.int32)

    gathered = _sc_gather(ids_flat, word_emb)

    aux = jnp.zeros((8, HIDDEN), jnp.float32)
    aux = aux.at[0].set(type_emb[0]).at[1].set(type_emb[1])
    aux = aux.at[2].set(ln_weight).at[3].set(ln_bias)

    out = _tc_ln(tt_flat, gathered, pos_emb, aux)
    return out.reshape(batch, seq, HIDDEN)


# speedup vs baseline: 1.5494x; 1.0109x over previous
import functools

import jax
import jax.numpy as jnp
from jax import lax
from jax.experimental import pallas as pl
from jax.experimental.pallas import tpu as pltpu
from jax.experimental.pallas import tpu_sc as plsc

HIDDEN = 768
MAX_POS = 2048
EPS = 1e-12

NC = 2
NS = 16
NW = NC * NS

CHUNK = 64

TOK_BLK = 256


def _sc_gather_body(ids_hbm, table_hbm, out_hbm, idx_v, rows_v, sem0, sem1):
    wid = lax.axis_index("s") * NC + lax.axis_index("c")
    n_chunks = ids_hbm.shape[0] // NW
    base_chunk = wid * n_chunks
    sems = (sem0, sem1)
    pltpu.sync_copy(ids_hbm.at[pl.ds(base_chunk, n_chunks)], idx_v)
    handles = [None, None]
    handles[0] = pltpu.async_copy(
        table_hbm.at[idx_v.at[0]], rows_v.at[0], sems[0])
    for j in range(n_chunks):
        cur = j % 2
        nxt = (j + 1) % 2
        if j + 1 < n_chunks:
            handles[nxt] = pltpu.async_copy(
                table_hbm.at[idx_v.at[j + 1]], rows_v.at[nxt], sems[nxt])
        handles[cur].wait()
        pltpu.sync_copy(
            rows_v.at[cur], out_hbm.at[pl.ds((base_chunk + j) * CHUNK, CHUNK)])


def _sc_gather(ids_flat, word_emb):
    tokens = ids_flat.shape[0]
    ids2d = ids_flat.reshape(tokens // CHUNK, CHUNK)
    n_chunks = (tokens // CHUNK) // NW
    mesh = plsc.VectorSubcoreMesh(core_axis_name="c", subcore_axis_name="s")
    return pl.kernel(
        _sc_gather_body,
        out_type=jax.ShapeDtypeStruct((tokens, HIDDEN), jnp.float32),
        mesh=mesh,
        scratch_types=[
            pltpu.VMEM((n_chunks, CHUNK), jnp.int32),
            pltpu.VMEM((2, CHUNK, HIDDEN), jnp.float32),
            pltpu.SemaphoreType.DMA,
            pltpu.SemaphoreType.DMA,
        ],
    )(ids2d, word_emb)


def _tc_ln_body(tt_ref, gath_ref, pos_ref, aux_ref, out_ref):
    tt = tt_ref[0][0].reshape(TOK_BLK, 1)
    type0 = aux_ref[0:1, :]
    type1 = aux_ref[1:2, :]
    w = aux_ref[2:3, :]
    b = aux_ref[3:4, :]
    e = gath_ref[...] + pos_ref[...] + jnp.where(tt == 0, type0, type1)
    mean = jnp.mean(e, axis=-1, keepdims=True)
    cen = e - mean
    var = jnp.mean(cen * cen, axis=-1, keepdims=True)
    out_ref[...] = w * (cen / jnp.sqrt(var + EPS)) + b


def _tc_ln(tt_flat, gathered, pos_emb, aux, batch):
    tokens = gathered.shape[0]
    n_blk = tokens // TOK_BLK
    seq_blocks = n_blk // batch
    tt3d = tt_flat.reshape(n_blk, 1, TOK_BLK)
    return pl.pallas_call(
        _tc_ln_body,
        grid=(seq_blocks, batch),
        in_specs=[
            pl.BlockSpec((1, 1, TOK_BLK), lambda i, j: (j * seq_blocks + i, 0, 0)),
            pl.BlockSpec((TOK_BLK, HIDDEN), lambda i, j: (j * seq_blocks + i, 0)),
            pl.BlockSpec((TOK_BLK, HIDDEN), lambda i, j: (i, 0)),
            pl.BlockSpec((8, HIDDEN), lambda i, j: (0, 0)),
        ],
        out_specs=pl.BlockSpec((TOK_BLK, HIDDEN), lambda i, j: (j * seq_blocks + i, 0)),
        out_shape=jax.ShapeDtypeStruct((tokens, HIDDEN), jnp.float32),
    )(tt3d, gathered, pos_emb, aux)


def kernel(input_ids, token_type_ids, word_emb, pos_emb, type_emb, ln_weight,
           ln_bias):
    batch, seq = input_ids.shape
    tokens = batch * seq
    ids_flat = input_ids.reshape(tokens).astype(jnp.int32)
    tt_flat = token_type_ids.reshape(tokens).astype(jnp.int32)

    gathered = _sc_gather(ids_flat, word_emb)

    aux = jnp.zeros((8, HIDDEN), jnp.float32)
    aux = aux.at[0].set(type_emb[0]).at[1].set(type_emb[1])
    aux = aux.at[2].set(ln_weight).at[3].set(ln_bias)

    out = _tc_ln(tt_flat, gathered, pos_emb, aux, batch)
    return out.reshape(batch, seq, HIDDEN)


# speedup vs baseline: 1.9412x; 1.2528x over previous
import functools

import jax
import jax.numpy as jnp
from jax import lax
from jax.experimental import pallas as pl
from jax.experimental.pallas import tpu as pltpu
from jax.experimental.pallas import tpu_sc as plsc

HIDDEN = 768
MAX_POS = 2048
EPS = 1e-12

NC = 2
NS = 16
NW = NC * NS

CHUNK = 64

TOK_BLK = 2048


def _sc_gather_body(ids_hbm, table_hbm, out_hbm, idx_v, rows_v, sem0, sem1):
    wid = lax.axis_index("s") * NC + lax.axis_index("c")
    n_chunks = ids_hbm.shape[0] // NW
    base_chunk = wid * n_chunks
    sems = (sem0, sem1)
    pltpu.sync_copy(ids_hbm.at[pl.ds(base_chunk, n_chunks)], idx_v)
    handles = [None, None]
    handles[0] = pltpu.async_copy(
        table_hbm.at[idx_v.at[0]], rows_v.at[0], sems[0])
    for j in range(n_chunks):
        cur = j % 2
        nxt = (j + 1) % 2
        if j + 1 < n_chunks:
            handles[nxt] = pltpu.async_copy(
                table_hbm.at[idx_v.at[j + 1]], rows_v.at[nxt], sems[nxt])
        handles[cur].wait()
        pltpu.sync_copy(
            rows_v.at[cur], out_hbm.at[pl.ds((base_chunk + j) * CHUNK, CHUNK)])


def _sc_gather(ids_flat, word_emb):
    tokens = ids_flat.shape[0]
    ids2d = ids_flat.reshape(tokens // CHUNK, CHUNK)
    n_chunks = (tokens // CHUNK) // NW
    mesh = plsc.VectorSubcoreMesh(core_axis_name="c", subcore_axis_name="s")
    return pl.kernel(
        _sc_gather_body,
        out_type=jax.ShapeDtypeStruct((tokens, HIDDEN), jnp.float32),
        mesh=mesh,
        scratch_types=[
            pltpu.VMEM((n_chunks, CHUNK), jnp.int32),
            pltpu.VMEM((2, CHUNK, HIDDEN), jnp.float32),
            pltpu.SemaphoreType.DMA,
            pltpu.SemaphoreType.DMA,
        ],
    )(ids2d, word_emb)


def _tc_ln_body(tt_ref, gath_ref, pos_ref, aux_ref, out_ref):
    tt = tt_ref[0][0].reshape(TOK_BLK, 1)
    type0 = aux_ref[0:1, :]
    type1 = aux_ref[1:2, :]
    w = aux_ref[2:3, :]
    b = aux_ref[3:4, :]
    e = gath_ref[...] + pos_ref[...] + jnp.where(tt == 0, type0, type1)
    mean = jnp.mean(e, axis=-1, keepdims=True)
    cen = e - mean
    var = jnp.mean(cen * cen, axis=-1, keepdims=True)
    out_ref[...] = w * (cen / jnp.sqrt(var + EPS)) + b


def _tc_ln(tt_flat, gathered, pos_emb, aux, batch):
    tokens = gathered.shape[0]
    n_blk = tokens // TOK_BLK
    seq_blocks = n_blk // batch
    tt3d = tt_flat.reshape(n_blk, 1, TOK_BLK)
    return pl.pallas_call(
        _tc_ln_body,
        grid=(seq_blocks, batch),
        in_specs=[
            pl.BlockSpec((1, 1, TOK_BLK), lambda i, j: (j * seq_blocks + i, 0, 0)),
            pl.BlockSpec((TOK_BLK, HIDDEN), lambda i, j: (j * seq_blocks + i, 0)),
            pl.BlockSpec((TOK_BLK, HIDDEN), lambda i, j: (i, 0)),
            pl.BlockSpec((8, HIDDEN), lambda i, j: (0, 0)),
        ],
        out_specs=pl.BlockSpec((TOK_BLK, HIDDEN), lambda i, j: (j * seq_blocks + i, 0)),
        out_shape=jax.ShapeDtypeStruct((tokens, HIDDEN), jnp.float32),
    )(tt3d, gathered, pos_emb, aux)


def kernel(input_ids, token_type_ids, word_emb, pos_emb, type_emb, ln_weight,
           ln_bias):
    batch, seq = input_ids.shape
    tokens = batch * seq
    ids_flat = input_ids.reshape(tokens).astype(jnp.int32)
    tt_flat = token_type_ids.reshape(tokens).astype(jnp.int32)

    gathered = _sc_gather(ids_flat, word_emb)

    aux = jnp.zeros((8, HIDDEN), jnp.float32)
    aux = aux.at[0].set(type_emb[0]).at[1].set(type_emb[1])
    aux = aux.at[2].set(ln_weight).at[3].set(ln_bias)

    out = _tc_ln(tt_flat, gathered, pos_emb, aux, batch)
    return out.reshape(batch, seq, HIDDEN)
